# baseline (speedup 1.0000x reference)
import numpy as np
import jax
import jax.numpy as jnp
from jax import lax
from jax.experimental import pallas as pl
from jax.experimental.pallas import tpu as pltpu

_ISIZE = 224
_PSIZE = 30
_rng = np.random.RandomState(0)
_X = int(_rng.randint(0, _ISIZE - _PSIZE))
_Y = int(_rng.randint(0, _ISIZE - _PSIZE))

_ROWS = 3 * _ISIZE * _ISIZE // 128
_K = 4
_GS = 8
_R = 3


def _canvas_kernel(p_ref, c_ref):
    c_ref[...] = jnp.zeros_like(c_ref)
    c_ref[:, :, _X:_X + _PSIZE, _Y:_Y + _PSIZE] = p_ref[...]


def _add_kernel(x_hbm, c_hbm, o_hbm, cvs, in_bufs, out_bufs,
                in_sems, out_sems, c_sem):
    ngroups = x_hbm.shape[0] // _GS
    n_chunks = ngroups * _K

    pltpu.make_async_copy(c_hbm, cvs, c_sem).start()
    pltpu.make_async_copy(c_hbm, cvs, c_sem).wait()

    def in_copy(c, b):
        g, k = divmod(c, _K)
        return pltpu.make_async_copy(
            x_hbm.at[pl.ds(g * _GS, _GS), k], in_bufs.at[b], in_sems.at[b])

    def out_copy(c, b):
        g, k = divmod(c, _K)
        return pltpu.make_async_copy(
            out_bufs.at[b], o_hbm.at[pl.ds(g * _GS, _GS), k], out_sems.at[b])

    for c in range(min(_R, n_chunks)):
        in_copy(c, c % _R).start(priority=0)
    for c in range(n_chunks):
        b = c % _R
        in_copy(c, b).wait()
        if c >= _R:
            out_copy(c - _R, b).wait()
        out_bufs[b] = in_bufs[b] + cvs[...]
        out_copy(c, b).start(priority=1)
        if c + _R < n_chunks:
            in_copy(c + _R, b).start(priority=0)
    for c in range(max(0, n_chunks - _R), n_chunks):
        out_copy(c, c % _R).wait()


def kernel(x, patch):
    B = x.shape[0]
    canvas = pl.pallas_call(
        _canvas_kernel,
        out_shape=jax.ShapeDtypeStruct((1, 3, _ISIZE, _ISIZE), x.dtype),
    )(patch)
    x4 = x.reshape(B // _K, _K, _ROWS, 128)
    c2 = canvas.reshape(1, _ROWS, 128)

    out = pl.pallas_call(
        _add_kernel,
        in_specs=[
            pl.BlockSpec(memory_space=pltpu.HBM),
            pl.BlockSpec(memory_space=pltpu.HBM),
        ],
        out_specs=pl.BlockSpec(memory_space=pltpu.HBM),
        out_shape=jax.ShapeDtypeStruct((B // _K, _K, _ROWS, 128), x.dtype),
        scratch_shapes=[
            pltpu.VMEM((1, _ROWS, 128), x.dtype),
            pltpu.VMEM((_R, _GS, _ROWS, 128), x.dtype),
            pltpu.VMEM((_R, _GS, _ROWS, 128), x.dtype),
            pltpu.SemaphoreType.DMA((_R,)),
            pltpu.SemaphoreType.DMA((_R,)),
            pltpu.SemaphoreType.DMA,
        ],
        compiler_params=pltpu.CompilerParams(
            vmem_limit_bytes=110 * 1024 * 1024),
    )(x4, c2)
    return out.reshape(x.shape)
